# baseline (speedup 1.0000x reference)
import functools

import jax
import jax.numpy as jnp
from jax import lax
from jax.experimental import pallas as pl
from jax.experimental.pallas import tpu as pltpu
from jax.experimental.pallas import tpu_sc as plsc

NUM_CHANNEL = 8
VOCAB = 100000
HIDDEN = 128
MBS = 4
SEQ = 2048

LANES = 16

_info = plsc.get_sparse_core_info()
NC = _info.num_cores
NS = _info.num_subcores
NW = NC * NS
SPW = SEQ // NW
CS = 32
NSK = SPW // CS
NCHUNK = MBS * NSK
GROW = CS * NUM_CHANNEL
IDW = 128
IDROWS = MBS * SEQ * NUM_CHANNEL // IDW
IROW_PB = SEQ * NUM_CHANNEL // IDW
IROW_PW = MBS * SPW * NUM_CHANNEL // IDW
RPC = GROW // IDW
GSPLIT = 4

_mesh = plsc.VectorSubcoreMesh(core_axis_name="c", subcore_axis_name="s")


@functools.partial(
    pl.kernel,
    mesh=_mesh,
    out_type=jax.ShapeDtypeStruct((SEQ, MBS, HIDDEN), jnp.float32),
    scratch_types=[
        pltpu.VMEM((IROW_PW, IDW), jnp.int32),
        pltpu.VMEM((IROW_PW, IDW), jnp.int32),
        pltpu.VMEM((2, GROW, HIDDEN), jnp.float32),
        pltpu.VMEM((2, CS, HIDDEN), jnp.float32),
        pltpu.SemaphoreType.DMA,
        pltpu.SemaphoreType.DMA,
        pltpu.SemaphoreType.DMA,
        pltpu.SemaphoreType.DMA,
        pltpu.SemaphoreType.DMA,
    ],
)
def _sc_embed(ids_hbm, tab_hbm, out_hbm, ids_raw, ids_v, gbuf, obuf,
              isem, g0, g1, o0, o1):
    wid = lax.axis_index("s") * NC + lax.axis_index("c")
    s0 = wid * SPW
    gsem = (g0, g1)
    osem = (o0, o1)

    rpb = NSK * RPC
    icopies = [
        pltpu.async_copy(
            ids_hbm.at[pl.ds(b * IROW_PB + wid * rpb, rpb)],
            ids_raw.at[pl.ds(b * rpb, rpb)],
            isem,
        )
        for b in range(MBS)
    ]

    pat = (lax.iota(jnp.int32, LANES) & (NUM_CHANNEL - 1)) * VOCAB

    def prep_rows(lo, n):
        def row_body(r, carry):
            for v in range(IDW // LANES):
                sl = pl.ds(v * LANES, LANES)
                ids_v[r, sl] = ids_raw[r, sl] + pat
            return carry
        lax.fori_loop(lo, lo + n, row_body, 0, unroll=False)

    def fire(k, j):
        n = GROW // GSPLIT
        return [
            pltpu.async_copy(
                tab_hbm.at[ids_v.at[k * RPC + (i * n) // IDW,
                                    pl.ds((i * n) % IDW, n)]],
                gbuf.at[j, pl.ds(i * n, n)],
                gsem[j],
            )
            for i in range(GSPLIT)
        ]

    icopies[0].wait()
    prep_rows(0, rpb)
    gcopies = [fire(0, 0), fire(1, 1)]
    for b in range(1, MBS):
        icopies[b].wait()
        prep_rows(b * rpb, rpb)

    scopies = [None, None]

    for k in range(NCHUNK):
        j = k % 2
        b, sk = k // NSK, k % NSK
        for cp in gcopies[j]:
            cp.wait()
        if k + 2 < NCHUNK:
            gcopies[j] = fire(k + 2, j)

        def pos_body(p, carry, _j=j):
            for h in range(HIDDEN // LANES):
                sl = pl.ds(h * LANES, LANES)
                acc = gbuf[_j, p * NUM_CHANNEL, sl]
                for c in range(1, NUM_CHANNEL):
                    acc = acc + gbuf[_j, p * NUM_CHANNEL + c, sl]
                obuf[_j, p, sl] = acc
            return carry

        lax.fori_loop(0, CS, pos_body, 0, unroll=False)

        if scopies[j] is not None:
            scopies[j].wait()
        scopies[j] = pltpu.async_copy(
            obuf.at[j], out_hbm.at[pl.ds(s0 + sk * CS, CS), b], osem[j]
        )

    for cp in scopies:
        if cp is not None:
            cp.wait()


def kernel(audio_ids, tables):
    ids2 = audio_ids.reshape(IDROWS, IDW)
    flat_tab = tables.reshape(NUM_CHANNEL * VOCAB, HIDDEN)
    return _sc_embed(ids2, flat_tab)
